# baseline (speedup 1.0000x reference)
import functools

import jax
import jax.numpy as jnp
from jax import lax
from jax.experimental import pallas as pl
from jax.experimental.pallas import tpu as pltpu

CB = 8


def _corr_body(emb_ref, w_ref, b_ref, out_ref):
    i = pl.program_id(0)

    @pl.when(i == 0)
    def _():
        out_ref[...] = jnp.zeros_like(out_ref)

    emb = emb_ref[...]
    wblk = w_ref[...]
    bblk = b_ref[...]
    e = lax.dot_general(emb, wblk, (((2,), (1,)), ((1,), (0,))),
                        preferred_element_type=jnp.float32)
    e = e + bblk[0][:, None, :]
    out_ref[...] += lax.dot_general(e, e, (((0,), (0,)), ((1,), (1,))),
                                    preferred_element_type=jnp.float32)


def _correlation(emb, w, b):
    B_, C_, T = emb.shape
    grid = (C_ // CB,)
    return pl.pallas_call(
        _corr_body,
        grid=grid,
        in_specs=[
            pl.BlockSpec((B_, CB, T), lambda i: (0, i, 0)),
            pl.BlockSpec((CB, T, T), lambda i: (i, 0, 0)),
            pl.BlockSpec((1, CB, T), lambda i: (0, i, 0)),
        ],
        out_specs=pl.BlockSpec((B_, T, T), lambda i: (0, 0, 0)),
        out_shape=jax.ShapeDtypeStruct((B_, T, T), jnp.float32),
        compiler_params=pltpu.CompilerParams(
            dimension_semantics=("arbitrary",),
        ),
    )(emb, w, b)


def kernel(im, weight, w, b):
    B_, C_, H_, W_ = im.shape
    T = w.shape[1]
    one_dim = H_ * W_
    x = im.reshape(B_, C_, one_dim)
    c = weight.reshape(B_, one_dim)
    top_vals, top_idx = jax.lax.top_k(c, T)
    kth = top_vals[:, -1][:, None]
    wsel = jax.nn.relu(jnp.floor(top_vals - kth) + 0.5) * 2.0
    idx3 = jnp.broadcast_to(top_idx[:, None, :], (B_, C_, T))
    img = jnp.take_along_axis(x, idx3, axis=2)
    emb = wsel[:, None, :] * img
    return _correlation(emb, w, b)


# speedup vs baseline: 1.0002x; 1.0002x over previous
import jax
import jax.numpy as jnp
from jax import lax
from jax.experimental import pallas as pl
from jax.experimental.pallas import tpu as pltpu

CB = 8


def _corr_body(emb_ref, w_ref, b_ref, out_ref):
    i = pl.program_id(0)

    @pl.when(i == 0)
    def _():
        out_ref[...] = jnp.zeros_like(out_ref)

    emb = emb_ref[...]
    wblk = w_ref[...]
    bblk = b_ref[...]
    e = lax.dot_general(emb, wblk, (((2,), (1,)), ((1,), (0,))),
                        preferred_element_type=jnp.float32)
    e = e + bblk[0][:, None, :]
    out_ref[...] += lax.dot_general(e, e, (((0,), (0,)), ((1,), (1,))),
                                    preferred_element_type=jnp.float32)


def _correlation(emb, w, b):
    B_, C_, T = emb.shape
    grid = (C_ // CB,)
    return pl.pallas_call(
        _corr_body,
        grid=grid,
        in_specs=[
            pl.BlockSpec((B_, CB, T), lambda i: (0, i, 0)),
            pl.BlockSpec((CB, T, T), lambda i: (i, 0, 0)),
            pl.BlockSpec((1, CB, T), lambda i: (0, i, 0)),
        ],
        out_specs=pl.BlockSpec((B_, T, T), lambda i: (0, 0, 0)),
        out_shape=jax.ShapeDtypeStruct((B_, T, T), jnp.float32),
        compiler_params=pltpu.CompilerParams(
            dimension_semantics=("arbitrary",),
        ),
    )(emb, w, b)


def kernel(im, weight, w, b):
    B_, C_, H_, W_ = im.shape
    T = w.shape[1]
    one_dim = H_ * W_
    x = im.reshape(B_, C_, one_dim)
    c = weight.reshape(B_, one_dim)
    top_vals, top_idx = jax.lax.top_k(c, T)
    kth = top_vals[:, -1][:, None]
    wsel = jax.nn.relu(jnp.floor(top_vals - kth) + 0.5) * 2.0
    idx3 = jnp.broadcast_to(top_idx[:, None, :], (B_, C_, T))
    img = jnp.take_along_axis(x, idx3, axis=2)
    emb = wsel[:, None, :] * img
    return _correlation(emb, w, b)


# speedup vs baseline: 1.5230x; 1.5226x over previous
import jax
import jax.numpy as jnp
from jax import lax
from jax.experimental import pallas as pl
from jax.experimental.pallas import tpu as pltpu

CB = 8


def _corr_body(emb_ref, w_ref, b_ref, out_ref):
    i = pl.program_id(0)

    @pl.when(i == 0)
    def _():
        out_ref[...] = jnp.zeros_like(out_ref)

    emb = emb_ref[...]
    wblk = w_ref[...]
    bblk = b_ref[...]
    e = lax.dot_general(emb, wblk, (((2,), (1,)), ((1,), (0,))),
                        preferred_element_type=jnp.float32)
    e = e + bblk[0][:, None, :]
    out_ref[...] += lax.dot_general(e, e, (((0,), (0,)), ((1,), (1,))),
                                    preferred_element_type=jnp.float32)


def _correlation(emb, w, b):
    B_, C_, T = emb.shape
    grid = (C_ // CB,)
    return pl.pallas_call(
        _corr_body,
        grid=grid,
        in_specs=[
            pl.BlockSpec((B_, CB, T), lambda i: (0, i, 0)),
            pl.BlockSpec((CB, T, T), lambda i: (i, 0, 0)),
            pl.BlockSpec((1, CB, T), lambda i: (0, i, 0)),
        ],
        out_specs=pl.BlockSpec((B_, T, T), lambda i: (0, 0, 0)),
        out_shape=jax.ShapeDtypeStruct((B_, T, T), jnp.float32),
        compiler_params=pltpu.CompilerParams(
            dimension_semantics=("arbitrary",),
        ),
    )(emb, w, b)


def kernel(im, weight, w, b):
    B_, C_, H_, W_ = im.shape
    T = w.shape[1]
    one_dim = H_ * W_
    x = im.reshape(B_, C_, one_dim)
    c = weight.reshape(B_, one_dim)
    G = 8
    S = one_dim // G
    v1, i1 = jax.lax.top_k(c.reshape(B_ * G, S), T)
    gbase = (jnp.arange(B_ * G, dtype=jnp.int32) % G * S)[:, None]
    gidx = (i1 + gbase).reshape(B_, G * T)
    v2, i2 = jax.lax.top_k(v1.reshape(B_, G * T), T)
    top_vals = v2
    top_idx = jnp.take_along_axis(gidx, i2, axis=1)
    kth = top_vals[:, -1][:, None]
    wsel = jax.nn.relu(jnp.floor(top_vals - kth) + 0.5) * 2.0
    idx3 = jnp.broadcast_to(top_idx[:, None, :], (B_, C_, T))
    img = jnp.take_along_axis(x, idx3, axis=2)
    emb = wsel[:, None, :] * img
    return _correlation(emb, w, b)
